# baseline (speedup 1.0000x reference)
import functools

import numpy as np
import jax
import jax.numpy as jnp
from jax import lax
from jax.experimental import pallas as pl
from jax.experimental.pallas import tpu as pltpu
from jax.experimental.pallas import tpu_sc as plsc

_C = 21
_NB = 1024
_CPAD = 24
_TRASH = 2 * _CPAD * _NB
_HROWS = 2 * _CPAD + 1
_HBINS = _HROWS * _NB
_B, _H, _W = 4, 512, 512
_HW = _H * _W
_P = _B * _HW
_NW = 32
_S = 1024
_CHUNKS_PER_W = _P // (_NW * _S)
_HB = 16
_PXB = _HB * _W
_JBLK = _HW // _PXB
_ROWS = _B * _JBLK * _CPAD
_MV = 12
_PAT = 16 * _MV
_SX = _S + 128


def _bins_body(x_ref, y_ref, out_ref):
    x = x_ref[0]
    y = y_ref[...]
    m = jnp.max(x, axis=0, keepdims=True)
    e = jnp.exp(x - m)
    s = jnp.sum(e, axis=0, keepdims=True)
    p = e / s
    nbf = jnp.float32(_NB)
    bneg = jnp.minimum((p * nbf).astype(jnp.int32), _NB - 1)
    bpos = jnp.minimum(((1.0 - p) * nbf).astype(jnp.int32), _NB - 1)
    cidx = lax.broadcasted_iota(jnp.int32, x.shape, 0)
    ispos = y == cidx
    val = cidx * _NB + jnp.where(ispos, _CPAD * _NB + bpos, bneg)
    ci = lax.broadcasted_iota(jnp.int32, (_CPAD - _C, _PXB), 0) + _C
    qi = lax.broadcasted_iota(jnp.int32, (_CPAD - _C, _PXB), 1)
    pad = _TRASH + ((ci + 8 * qi) & 15)
    out_ref[...] = jnp.concatenate([val.reshape(_C, _PXB), pad], axis=0)


def _cumsum_lanes(x):
    n = x.shape[-1]
    s = 1
    while s < n:
        shifted = jnp.concatenate(
            [jnp.zeros(x.shape[:-1] + (s,), x.dtype), x[..., :-s]], axis=-1)
        x = x + shifted
        s *= 2
    return x


def _fin_body(h1_ref, h2_ref, out_ref):
    h = (h1_ref[...] + h2_ref[...]).reshape(_NW, _HROWS, _NB)
    hsum = jnp.sum(h, axis=0)
    neg = hsum[:_CPAD]
    pos = hsum[_CPAD:2 * _CPAD]
    allh = neg + pos
    t_tot = jnp.sum(pos, axis=-1, keepdims=True)
    n_tot = jnp.sum(allh, axis=-1, keepdims=True)
    pall = _cumsum_lanes(allh)
    ppos = _cumsum_lanes(pos)
    n_pre = n_tot - pall
    k_pre = t_tot - ppos
    n_post = n_pre + allh
    k_post = k_pre + pos

    def jacc(n, k):
        return 1.0 - (t_tot - k) / jnp.maximum(t_tot + n - k, 1.0)

    r = (lax.broadcasted_iota(jnp.int32, (_CPAD, _NB), 1).astype(jnp.float32)
         + 0.5) * (1.0 / _NB)
    loss_c = jnp.sum(r * (jacc(n_post, k_post) - jacc(n_pre, k_pre)),
                     axis=-1, keepdims=True)
    pres = (t_tot > 0).astype(jnp.float32)
    num = jnp.sum(loss_c * pres, axis=0, keepdims=True)
    den = jnp.sum(pres, axis=0, keepdims=True)
    out_ref[...] = num / jnp.maximum(den, 1.0)


_sc_mesh = plsc.VectorSubcoreMesh(core_axis_name="c", subcore_axis_name="s")


@functools.partial(
    pl.kernel,
    mesh=_sc_mesh,
    compiler_params=pltpu.CompilerParams(needs_layout_passes=False),
    out_type=jax.ShapeDtypeStruct((_NW * _HBINS,), jnp.float32),
    scratch_types=[
        pltpu.VMEM((_CPAD, _SX), jnp.int32),
        pltpu.VMEM((_CPAD, _SX), jnp.int32),
        pltpu.VMEM((_PAT,), jnp.int32),
        pltpu.VMEM((_PAT,), jnp.int32),
        pltpu.VMEM((_HBINS,), jnp.float32),
        pltpu.SemaphoreType.DMA,
        pltpu.SemaphoreType.DMA,
    ],
)
def _sc_hist(bins_hbm, prow_hbm, pcol_hbm, out_hbm, buf_a, buf_b,
             prow_v, pcol_v, hist_v, sem_a, sem_b):
    wid = lax.axis_index("s") * 2 + lax.axis_index("c")
    pltpu.sync_copy(prow_hbm, prow_v)
    pltpu.sync_copy(pcol_hbm, pcol_v)

    zero16 = jnp.zeros((16,), jnp.float32)

    def zbody(i, carry):
        hist_v[pl.ds(i * 16, 16)] = zero16
        return carry

    lax.fori_loop(0, _HBINS // 16, zbody, 0)

    ones16 = jnp.ones((16,), jnp.float32)
    g0 = wid * (_CHUNKS_PER_W // 2)

    def fire(ci, buf, sem):
        g = g0 + ci
        rr = pl.ds((g // 8) * _CPAD, _CPAD)
        c0 = (g % 8) * _S
        pltpu.async_copy(bins_hbm.at[rr, pl.ds(c0, _S)],
                         buf.at[:, pl.ds(0, _S)], sem)
        pltpu.async_copy(bins_hbm.at[rr, pl.ds(c0, 128)],
                         buf.at[:, pl.ds(_S, 128)], sem)

    def drain(buf, sem):
        pltpu.make_async_copy(bins_hbm.at[pl.ds(0, _CPAD), pl.ds(0, _S)],
                              buf.at[:, pl.ds(0, _S)], sem).wait()
        pltpu.make_async_copy(bins_hbm.at[pl.ds(0, _CPAD), pl.ds(0, 128)],
                              buf.at[:, pl.ds(_S, 128)], sem).wait()

    rowp = [prow_v[pl.ds(k * 16, 16)] for k in range(3)]
    colp = [pcol_v[pl.ds(k * 16, 16)] for k in range(_MV)]

    def compute(buf):
        ivs0 = tuple(colp)

        def gbody(g, ivs):
            vals = [plsc.load_gather(buf, [rowp[m % 3], ivs[m]])
                    for m in range(_MV)]
            for m in range(_MV):
                plsc.addupdate_scatter(hist_v, [vals[m]], ones16)
            return tuple(iv + (16 * _MV // 24) for iv in ivs)

        lax.fori_loop(0, _CPAD * _S // (16 * _MV), gbody, ivs0)

    fire(0, buf_a, sem_a)
    half_chunks = _CHUNKS_PER_W // 2

    def pair_body(i, carry):
        ci = i * 2
        fire(ci + 1, buf_b, sem_b)
        drain(buf_a, sem_a)
        compute(buf_a)

        @pl.when(ci + 2 < half_chunks)
        def _():
            fire(ci + 2, buf_a, sem_a)

        drain(buf_b, sem_b)
        compute(buf_b)
        return carry

    lax.fori_loop(0, half_chunks // 2, pair_body, 0)
    pltpu.sync_copy(hist_v, out_hbm.at[pl.ds(wid * _HBINS, _HBINS)])


_t = np.arange(_PAT)
_PROW = np.asarray(_t % _CPAD, dtype=np.int32)
_PCOL = np.asarray(_t // _CPAD + _t % _CPAD, dtype=np.int32)


def _bins_half(x, y):
    return pl.pallas_call(
        _bins_body,
        grid=(_B // 2, _JBLK),
        in_specs=[
            pl.BlockSpec((1, _C, _HB, _W), lambda b, j: (b, 0, j, 0)),
            pl.BlockSpec((1, _HB, _W), lambda b, j: (b, j, 0)),
        ],
        out_specs=pl.BlockSpec((_CPAD, _PXB), lambda b, j: (b * _JBLK + j, 0)),
        out_shape=jax.ShapeDtypeStruct((_ROWS // 2, _PXB), jnp.int32),
    )(x, y)


def kernel(inputs, targets):
    y = targets.astype(jnp.int32)
    prow = jnp.asarray(_PROW)
    pcol = jnp.asarray(_PCOL)

    bins1 = _bins_half(inputs[:2], y[:2])
    parts1 = _sc_hist(bins1, prow, pcol)
    bins2 = _bins_half(inputs[2:], y[2:])
    parts2 = _sc_hist(bins2, prow, pcol)

    out = pl.pallas_call(
        _fin_body,
        out_shape=jax.ShapeDtypeStruct((1, 1), jnp.float32),
    )(parts1.reshape(_NW * _HROWS, _NB), parts2.reshape(_NW * _HROWS, _NB))
    return out[0, 0]


# speedup vs baseline: 1.0760x; 1.0760x over previous
import functools

import numpy as np
import jax
import jax.numpy as jnp
from jax import lax
from jax.experimental import pallas as pl
from jax.experimental.pallas import tpu as pltpu
from jax.experimental.pallas import tpu_sc as plsc

_C = 21
_NB = 1024
_CPAD = 24
_TRASH = 2 * _CPAD * _NB
_HROWS = 2 * _CPAD + 1
_HBINS = _HROWS * _NB
_B, _H, _W = 4, 512, 512
_HW = _H * _W
_P = _B * _HW
_NW = 32
_S = 1024
_CHUNKS_PER_W = _P // (_NW * _S)
_HB = 16
_PXB = _HB * _W
_JBLK = _HW // _PXB
_ROWS = _B * _JBLK * _CPAD
_MV = 12
_PAT = 16 * _MV
_SX = _S + 128


def _bins_body(x_ref, y_ref, out_ref):
    x = x_ref[0]
    y = y_ref[...]
    m = jnp.max(x, axis=0, keepdims=True)
    e = jnp.exp(x - m)
    s = jnp.sum(e, axis=0, keepdims=True)
    p = e / s
    nbf = jnp.float32(_NB)
    bneg = jnp.minimum((p * nbf).astype(jnp.int32), _NB - 1)
    bpos = jnp.minimum(((1.0 - p) * nbf).astype(jnp.int32), _NB - 1)
    cidx = lax.broadcasted_iota(jnp.int32, x.shape, 0)
    ispos = y == cidx
    val = cidx * _NB + jnp.where(ispos, _CPAD * _NB + bpos, bneg)
    ci = lax.broadcasted_iota(jnp.int32, (_CPAD - _C, _PXB), 0) + _C
    qi = lax.broadcasted_iota(jnp.int32, (_CPAD - _C, _PXB), 1)
    pad = _TRASH + ((ci + 8 * qi) & 15)
    out_ref[...] = jnp.concatenate([val.reshape(_C, _PXB), pad], axis=0)


def _cumsum_lanes(x):
    n = x.shape[-1]
    s = 1
    while s < n:
        shifted = jnp.concatenate(
            [jnp.zeros(x.shape[:-1] + (s,), x.dtype), x[..., :-s]], axis=-1)
        x = x + shifted
        s *= 2
    return x


def _fin_body(h_ref, out_ref):
    h = h_ref[...].reshape(_NW, _HROWS, _NB)
    hsum = jnp.sum(h, axis=0)
    neg = hsum[:_CPAD]
    pos = hsum[_CPAD:2 * _CPAD]
    allh = neg + pos
    t_tot = jnp.sum(pos, axis=-1, keepdims=True)
    n_tot = jnp.sum(allh, axis=-1, keepdims=True)
    pall = _cumsum_lanes(allh)
    ppos = _cumsum_lanes(pos)
    n_pre = n_tot - pall
    k_pre = t_tot - ppos
    n_post = n_pre + allh
    k_post = k_pre + pos

    def jacc(n, k):
        return 1.0 - (t_tot - k) / jnp.maximum(t_tot + n - k, 1.0)

    r = (lax.broadcasted_iota(jnp.int32, (_CPAD, _NB), 1).astype(jnp.float32)
         + 0.5) * (1.0 / _NB)
    loss_c = jnp.sum(r * (jacc(n_post, k_post) - jacc(n_pre, k_pre)),
                     axis=-1, keepdims=True)
    pres = (t_tot > 0).astype(jnp.float32)
    num = jnp.sum(loss_c * pres, axis=0, keepdims=True)
    den = jnp.sum(pres, axis=0, keepdims=True)
    out_ref[...] = num / jnp.maximum(den, 1.0)


_sc_mesh = plsc.VectorSubcoreMesh(core_axis_name="c", subcore_axis_name="s")


@functools.partial(
    pl.kernel,
    mesh=_sc_mesh,
    compiler_params=pltpu.CompilerParams(needs_layout_passes=False),
    out_type=jax.ShapeDtypeStruct((_NW * _HBINS,), jnp.float32),
    scratch_types=[
        pltpu.VMEM((_CPAD, _SX), jnp.int32),
        pltpu.VMEM((_CPAD, _SX), jnp.int32),
        pltpu.VMEM((_PAT,), jnp.int32),
        pltpu.VMEM((_PAT,), jnp.int32),
        pltpu.VMEM((_HBINS,), jnp.float32),
        pltpu.SemaphoreType.DMA,
        pltpu.SemaphoreType.DMA,
    ],
)
def _sc_hist(bins_hbm, prow_hbm, pcol_hbm, out_hbm, buf_a, buf_b,
             prow_v, pcol_v, hist_v, sem_a, sem_b):
    wid = lax.axis_index("s") * 2 + lax.axis_index("c")
    pltpu.sync_copy(prow_hbm, prow_v)
    pltpu.sync_copy(pcol_hbm, pcol_v)

    zero16 = jnp.zeros((16,), jnp.float32)

    def zbody(i, carry):
        hist_v[pl.ds(i * 16, 16)] = zero16
        return carry

    lax.fori_loop(0, _HBINS // 16, zbody, 0)

    ones16 = jnp.ones((16,), jnp.float32)
    g0 = wid * _CHUNKS_PER_W

    def fire(ci, buf, sem):
        g = g0 + ci
        rr = pl.ds((g // 8) * _CPAD, _CPAD)
        c0 = (g % 8) * _S
        pltpu.async_copy(bins_hbm.at[rr, pl.ds(c0, _S)],
                         buf.at[:, pl.ds(0, _S)], sem)
        pltpu.async_copy(bins_hbm.at[rr, pl.ds(c0, 128)],
                         buf.at[:, pl.ds(_S, 128)], sem)

    def drain(buf, sem):
        pltpu.make_async_copy(bins_hbm.at[pl.ds(0, _CPAD), pl.ds(0, _S)],
                              buf.at[:, pl.ds(0, _S)], sem).wait()
        pltpu.make_async_copy(bins_hbm.at[pl.ds(0, _CPAD), pl.ds(0, 128)],
                              buf.at[:, pl.ds(_S, 128)], sem).wait()

    rowp = [prow_v[pl.ds(k * 16, 16)] for k in range(3)]
    colp = [pcol_v[pl.ds(k * 16, 16)] for k in range(_MV)]

    def compute(buf):
        ivs0 = tuple(colp)

        def gbody(g, ivs):
            vals = [plsc.load_gather(buf, [rowp[m % 3], ivs[m]])
                    for m in range(_MV)]
            for m in range(_MV):
                plsc.addupdate_scatter(hist_v, [vals[m]], ones16)
            return tuple(iv + (16 * _MV // 24) for iv in ivs)

        lax.fori_loop(0, _CPAD * _S // (16 * _MV), gbody, ivs0)

    fire(0, buf_a, sem_a)

    def pair_body(i, carry):
        ci = i * 2
        fire(ci + 1, buf_b, sem_b)
        drain(buf_a, sem_a)
        compute(buf_a)

        @pl.when(ci + 2 < _CHUNKS_PER_W)
        def _():
            fire(ci + 2, buf_a, sem_a)

        drain(buf_b, sem_b)
        compute(buf_b)
        return carry

    lax.fori_loop(0, _CHUNKS_PER_W // 2, pair_body, 0)
    pltpu.sync_copy(hist_v, out_hbm.at[pl.ds(wid * _HBINS, _HBINS)])


_t = np.arange(_PAT)
_PROW = np.asarray(_t % _CPAD, dtype=np.int32)
_PCOL = np.asarray(_t // _CPAD + _t % _CPAD, dtype=np.int32)


def kernel(inputs, targets):
    y = targets.astype(jnp.int32)

    bins = pl.pallas_call(
        _bins_body,
        grid=(_B, _JBLK),
        in_specs=[
            pl.BlockSpec((1, _C, _HB, _W), lambda b, j: (b, 0, j, 0)),
            pl.BlockSpec((1, _HB, _W), lambda b, j: (b, j, 0)),
        ],
        out_specs=pl.BlockSpec((_CPAD, _PXB), lambda b, j: (b * _JBLK + j, 0)),
        out_shape=jax.ShapeDtypeStruct((_ROWS, _PXB), jnp.int32),
    )(inputs, y)

    parts = _sc_hist(bins, jnp.asarray(_PROW), jnp.asarray(_PCOL))

    out = pl.pallas_call(
        _fin_body,
        out_shape=jax.ShapeDtypeStruct((1, 1), jnp.float32),
    )(parts.reshape(_NW * _HROWS, _NB))
    return out[0, 0]
